# baseline (speedup 1.0000x reference)
import functools

import jax
import jax.numpy as jnp
from jax import lax
from jax.experimental import pallas as pl
from jax.experimental.pallas import tpu as pltpu
from jax.experimental.pallas import tpu_sc as plsc

VOCAB = 1000000
EMBED_DIM = 64
SEQ_LEN = 200
BATCH = 4096

NUM_CORES = 2
NUM_SUBCORES = 16
NUM_WORKERS = NUM_CORES * NUM_SUBCORES
BLK = BATCH // NUM_WORKERS
NBUF = 4
NSPLIT = 2

_mesh = plsc.VectorSubcoreMesh(core_axis_name="c", subcore_axis_name="s")


def _make_embed(nseq):
    assert nseq >= 2 * NBUF

    @functools.partial(
        pl.kernel,
        mesh=_mesh,
        out_type=jax.ShapeDtypeStruct((nseq, BATCH, EMBED_DIM), jnp.float32),
        scratch_types=[
            pltpu.VMEM((nseq, BLK), jnp.int32),
            [pltpu.VMEM((BLK, EMBED_DIM), jnp.float32) for _ in range(NBUF)],
            [pltpu.SemaphoreType.DMA for _ in range(NBUF)],
            [pltpu.SemaphoreType.DMA for _ in range(NBUF)],
        ],
        compiler_params=pltpu.CompilerParams(use_tc_tiling_on_sc=False),
    )
    def _embed(ids_t_hbm, wt_hbm, pos_rep_hbm, out_hbm, idx_all, rows,
               gsem, ssem):
        wid = lax.axis_index("s") * NUM_CORES + lax.axis_index("c")
        pltpu.sync_copy(ids_t_hbm.at[:, pl.ds(wid * BLK, BLK)], idx_all)

        def prep_start(s, b):
            pltpu.sync_copy(pos_rep_hbm.at[s], rows[b])
            pltpu.async_copy(wt_hbm.at[idx_all.at[s]], rows[b], gsem[b],
                             add=True)

        def gather_wait(s, b):
            pltpu.make_async_copy(
                wt_hbm.at[idx_all.at[s]], rows[b], gsem[b]).wait()

        def out_slice(s):
            return out_hbm.at[s, pl.ds(wid * BLK, BLK)]

        def store_start(s, b):
            pltpu.async_copy(rows[b], out_slice(s), ssem[b])

        def store_wait(s, b):
            pltpu.make_async_copy(rows[b], out_slice(s), ssem[b]).wait()

        for k in range(NBUF - 1):
            prep_start(k, k)

        for s in range(NBUF):
            b = s % NBUF
            gather_wait(s, b)
            store_start(s, b)
            nb = (b + NBUF - 1) % NBUF
            if s > 0:
                store_wait(s - 1, nb)
            prep_start(s + NBUF - 1, nb)

        def full_step(s, b):
            gather_wait(s, b)
            store_start(s, b)
            nb = (b + NBUF - 1) % NBUF
            store_wait(s - 1, nb)
            prep_start(s + NBUF - 1, nb)

        def group_body(i, carry):
            for k in range(NBUF):
                s = NBUF * (i + 1) + k
                full_step(s, k)
            return carry

        n_full = nseq - 2 * NBUF
        lax.fori_loop(0, n_full // NBUF, group_body, 0)
        for r in range(n_full % NBUF):
            s = NBUF + (n_full // NBUF) * NBUF + r
            full_step(s, s % NBUF)

        s_last = nseq - NBUF
        full_step(s_last, s_last % NBUF)

        for s in range(nseq - NBUF + 1, nseq):
            b = s % NBUF
            gather_wait(s, b)
            store_start(s, b)
            store_wait(s - 1, (b + NBUF - 1) % NBUF)

        store_wait(nseq - 1, (nseq - 1) % NBUF)

    return _embed


_embed_chunk = _make_embed(SEQ_LEN // NSPLIT)


def kernel(input_ids, word_table, pos_table):
    ids_t = input_ids.T.astype(jnp.int32)
    pos_rep = jnp.broadcast_to(pos_table[:, None, :], (SEQ_LEN, BLK, EMBED_DIM))
    h = SEQ_LEN // NSPLIT
    parts = []
    for c in range(NSPLIT):
        out_t = _embed_chunk(ids_t[c * h:(c + 1) * h], word_table,
                             pos_rep[c * h:(c + 1) * h])
        parts.append(out_t.transpose(1, 0, 2))
    return jnp.concatenate(parts, axis=1)


# speedup vs baseline: 1.4122x; 1.4122x over previous
import functools

import jax
import jax.numpy as jnp
from jax import lax
from jax.experimental import pallas as pl
from jax.experimental.pallas import tpu as pltpu
from jax.experimental.pallas import tpu_sc as plsc

VOCAB = 1000000
EMBED_DIM = 64
SEQ_LEN = 200
BATCH = 4096

NUM_CORES = 2
NUM_SUBCORES = 16
NUM_WORKERS = NUM_CORES * NUM_SUBCORES
BLK = BATCH // NUM_WORKERS
NBUF = 4
NSPLIT = 1

_mesh = plsc.VectorSubcoreMesh(core_axis_name="c", subcore_axis_name="s")


def _make_embed(nseq):
    assert nseq >= 2 * NBUF

    @functools.partial(
        pl.kernel,
        mesh=_mesh,
        out_type=jax.ShapeDtypeStruct((nseq, BATCH, EMBED_DIM), jnp.float32),
        scratch_types=[
            pltpu.VMEM((nseq, BLK), jnp.int32),
            [pltpu.VMEM((BLK, EMBED_DIM), jnp.float32) for _ in range(NBUF)],
            [pltpu.SemaphoreType.DMA for _ in range(NBUF)],
            [pltpu.SemaphoreType.DMA for _ in range(NBUF)],
        ],
        compiler_params=pltpu.CompilerParams(use_tc_tiling_on_sc=False),
    )
    def _embed(ids_t_hbm, wt_hbm, pos_rep_hbm, out_hbm, idx_all, rows,
               gsem, ssem):
        wid = lax.axis_index("s") * NUM_CORES + lax.axis_index("c")
        pltpu.sync_copy(ids_t_hbm.at[:, pl.ds(wid * BLK, BLK)], idx_all)

        def prep_start(s, b):
            pltpu.sync_copy(pos_rep_hbm.at[s], rows[b])
            pltpu.async_copy(wt_hbm.at[idx_all.at[s]], rows[b], gsem[b],
                             add=True)

        def gather_wait(s, b):
            pltpu.make_async_copy(
                wt_hbm.at[idx_all.at[s]], rows[b], gsem[b]).wait()

        def out_slice(s):
            return out_hbm.at[s, pl.ds(wid * BLK, BLK)]

        def store_start(s, b):
            pltpu.async_copy(rows[b], out_slice(s), ssem[b])

        def store_wait(s, b):
            pltpu.make_async_copy(rows[b], out_slice(s), ssem[b]).wait()

        for k in range(NBUF - 1):
            prep_start(k, k)

        for s in range(NBUF):
            b = s % NBUF
            gather_wait(s, b)
            store_start(s, b)
            nb = (b + NBUF - 1) % NBUF
            if s > 0:
                store_wait(s - 1, nb)
            prep_start(s + NBUF - 1, nb)

        def full_step(s, b):
            gather_wait(s, b)
            store_start(s, b)
            nb = (b + NBUF - 1) % NBUF
            store_wait(s - 1, nb)
            prep_start(s + NBUF - 1, nb)

        def group_body(i, carry):
            for k in range(NBUF):
                s = NBUF * (i + 1) + k
                full_step(s, k)
            return carry

        n_full = nseq - 2 * NBUF
        lax.fori_loop(0, n_full // NBUF, group_body, 0)
        for r in range(n_full % NBUF):
            s = NBUF + (n_full // NBUF) * NBUF + r
            full_step(s, s % NBUF)

        s_last = nseq - NBUF
        full_step(s_last, s_last % NBUF)

        for s in range(nseq - NBUF + 1, nseq):
            b = s % NBUF
            gather_wait(s, b)
            store_start(s, b)
            store_wait(s - 1, (b + NBUF - 1) % NBUF)

        store_wait(nseq - 1, (nseq - 1) % NBUF)

    return _embed


_embed_chunk = _make_embed(SEQ_LEN // NSPLIT)


def kernel(input_ids, word_table, pos_table):
    ids_t = input_ids.T.astype(jnp.int32)
    pos_rep = jnp.broadcast_to(pos_table[:, None, :], (SEQ_LEN, BLK, EMBED_DIM))
    out_t = _embed_chunk(ids_t, word_table, pos_rep)
    return out_t.transpose(1, 0, 2)
